# baseline (speedup 1.0000x reference)
import functools

import jax
import jax.numpy as jnp
from jax import lax
from jax.experimental import pallas as pl
from jax.experimental.pallas import tpu as pltpu
from jax.experimental.pallas import tpu_sc as plsc

TOKENS = 8192
EMB = 2048
E = 16
BEST_K = 2
BLK = 512

NC = 2
NS = 16
NW = NC * NS
TPW = TOKENS // NW


def _logits_t_kernel(wb_ref, x_ref, bb_ref, out_ref):
    out_ref[...] = lax.dot_general(
        wb_ref[...], x_ref[...],
        dimension_numbers=(((1,), (1,)), ((), ())),
        preferred_element_type=jnp.float32) + bb_ref[...]


def _tc_logits_t(x, wb, bbc):
    return pl.pallas_call(
        _logits_t_kernel,
        grid=(TOKENS // BLK,),
        in_specs=[
            pl.BlockSpec((E, EMB), lambda i: (0, 0)),
            pl.BlockSpec((BLK, EMB), lambda i: (i, 0)),
            pl.BlockSpec((E, 1), lambda i: (0, 0)),
        ],
        out_specs=pl.BlockSpec((E, BLK), lambda i: (0, i)),
        out_shape=jax.ShapeDtypeStruct((E, TOKENS), jnp.float32),
    )(wb, x, bbc)


def _sc_router(lg_t):
    mesh = plsc.VectorSubcoreMesh(core_axis_name="c", subcore_axis_name="s",
                                  num_cores=NC)

    @functools.partial(
        pl.kernel,
        mesh=mesh,
        out_type=[
            jax.ShapeDtypeStruct((TOKENS * E,), jnp.float32),
            jax.ShapeDtypeStruct((TOKENS * BEST_K,), jnp.int32),
        ],
        scratch_types=[
            pltpu.VMEM((E, TPW), jnp.float32),
            pltpu.VMEM((TPW * E,), jnp.float32),
            pltpu.VMEM((TPW * BEST_K,), jnp.int32),
            pltpu.SemaphoreType.DMA,
            pltpu.SemaphoreType.DMA,
        ],
    )
    def k(lg_hbm, out_hbm, idx_hbm, lg_v, out_v, idx_v, sem_o, sem_i):
        wid = lax.axis_index("s") * NC + lax.axis_index("c")
        pltpu.sync_copy(lg_hbm.at[:, pl.ds(wid * TPW, TPW)], lg_v)
        lane = lax.iota(jnp.int32, E)

        def gat(t, idx):
            return t.at[idx].get(mode="promise_in_bounds")

        def transpose16(vs):
            for s in (1, 2, 4, 8):
                keep = (lane & s) == 0
                new = list(vs)
                for i in range(E):
                    if i & s == 0:
                        a, b = vs[i], vs[i | s]
                        new[i] = jnp.where(keep, a, gat(b, lane ^ s))
                        new[i | s] = jnp.where(keep, gat(a, lane ^ s), b)
                vs = new
            return vs

        def tile_body(g, _):
            t0 = lg_v[0, pl.ds(g * E, E)]
            m1 = t0
            i1 = jnp.zeros((E,), jnp.int32)
            m2 = jnp.full((E,), -jnp.inf)
            i2 = jnp.zeros((E,), jnp.int32)
            for e in range(1, E):
                te = lg_v[e, pl.ds(g * E, E)]
                c1 = te > m1
                c2 = te > m2
                m2 = jnp.where(c1, m1, jnp.where(c2, te, m2))
                i2 = jnp.where(c1, i1, jnp.where(c2, e, i2))
                m1 = jnp.where(c1, te, m1)
                i1 = jnp.where(c1, e, i1)
            e2 = jnp.exp(m2 - m1)
            rden = 1.0 / (1.0 + e2)
            p1 = rden
            p2 = e2 * rden
            rows_t = [jnp.where(i1 == e, p1,
                                jnp.where(i2 == e, p2, 0.0))
                      for e in range(E)]
            rows = transpose16(rows_t)
            for t in range(E):
                out_v[pl.ds((g * E + t) * E, E)] = rows[t]
            half = lane >> 1
            even = (lane & 1) == 0
            lo = jnp.where(even, gat(i1, half), gat(i2, half))
            hi = jnp.where(even, gat(i1, 8 + half), gat(i2, 8 + half))
            idx_v[pl.ds(g * 2 * E, E)] = lo
            idx_v[pl.ds((g * 2 + 1) * E, E)] = hi
            return _

        lax.fori_loop(0, TPW // E, tile_body, None)
        co = pltpu.async_copy(
            out_v, out_hbm.at[pl.ds(wid * (TPW * E), TPW * E)], sem_o)
        ci = pltpu.async_copy(
            idx_v, idx_hbm.at[pl.ds(wid * (TPW * BEST_K), TPW * BEST_K)],
            sem_i)
        co.wait()
        ci.wait()

    return k(lg_t)


@jax.jit
def kernel(x, Wb, bb, Wn, bn):
    del Wn, bn
    lg_t = _tc_logits_t(x, Wb, bb.reshape(E, 1))
    out_flat, idx_flat = _sc_router(lg_t)
    return (out_flat.reshape(TOKENS, E), idx_flat.reshape(TOKENS, BEST_K))


# speedup vs baseline: 1.0655x; 1.0655x over previous
import functools

import jax
import jax.numpy as jnp
from jax import lax
from jax.experimental import pallas as pl
from jax.experimental.pallas import tpu as pltpu
from jax.experimental.pallas import tpu_sc as plsc

TOKENS = 8192
EMB = 2048
E = 16
BEST_K = 2
BLK = 1024

NC = 2
NS = 16
NW = NC * NS
TPW = TOKENS // NW


def _logits_t_kernel(wb_ref, x_ref, bb_ref, out_ref):
    out_ref[...] = lax.dot_general(
        wb_ref[...], x_ref[...],
        dimension_numbers=(((1,), (1,)), ((), ())),
        preferred_element_type=jnp.float32) + bb_ref[...]


def _tc_logits_t(x, wb, bbc):
    return pl.pallas_call(
        _logits_t_kernel,
        grid=(TOKENS // BLK,),
        in_specs=[
            pl.BlockSpec((E, EMB), lambda i: (0, 0)),
            pl.BlockSpec((BLK, EMB), lambda i: (i, 0)),
            pl.BlockSpec((E, 1), lambda i: (0, 0)),
        ],
        out_specs=pl.BlockSpec((E, BLK), lambda i: (0, i)),
        out_shape=jax.ShapeDtypeStruct((E, TOKENS), jnp.float32),
    )(wb, x, bbc)


def _sc_router(lg_t):
    mesh = plsc.VectorSubcoreMesh(core_axis_name="c", subcore_axis_name="s",
                                  num_cores=NC)

    @functools.partial(
        pl.kernel,
        mesh=mesh,
        out_type=[
            jax.ShapeDtypeStruct((TOKENS * E,), jnp.float32),
            jax.ShapeDtypeStruct((TOKENS * BEST_K,), jnp.int32),
        ],
        scratch_types=[
            pltpu.VMEM((E, TPW), jnp.float32),
            pltpu.VMEM((TPW * E,), jnp.float32),
            pltpu.VMEM((TPW * BEST_K,), jnp.int32),
            pltpu.SemaphoreType.DMA,
            pltpu.SemaphoreType.DMA,
        ],
    )
    def k(lg_hbm, out_hbm, idx_hbm, lg_v, out_v, idx_v, sem_o, sem_i):
        wid = lax.axis_index("s") * NC + lax.axis_index("c")
        pltpu.sync_copy(lg_hbm.at[:, pl.ds(wid * TPW, TPW)], lg_v)
        lane = lax.iota(jnp.int32, E)

        def gat(t, idx):
            return t.at[idx].get(mode="promise_in_bounds")

        def transpose16(vs):
            for s in (1, 2, 4, 8):
                keep = (lane & s) == 0
                new = list(vs)
                for i in range(E):
                    if i & s == 0:
                        a, b = vs[i], vs[i | s]
                        new[i] = jnp.where(keep, a, gat(b, lane ^ s))
                        new[i | s] = jnp.where(keep, gat(a, lane ^ s), b)
                vs = new
            return vs

        def tile_body(g, _):
            t0 = lg_v[0, pl.ds(g * E, E)]
            m1 = t0
            i1 = jnp.zeros((E,), jnp.int32)
            m2 = jnp.full((E,), -jnp.inf)
            i2 = jnp.zeros((E,), jnp.int32)
            for e in range(1, E):
                te = lg_v[e, pl.ds(g * E, E)]
                c1 = te > m1
                c2 = te > m2
                m2 = jnp.where(c1, m1, jnp.where(c2, te, m2))
                i2 = jnp.where(c1, i1, jnp.where(c2, e, i2))
                m1 = jnp.where(c1, te, m1)
                i1 = jnp.where(c1, e, i1)
            e2 = jnp.exp(m2 - m1)
            rden = 1.0 / (1.0 + e2)
            p1 = rden
            p2 = e2 * rden
            rows_t = [jnp.where(i1 == e, p1,
                                jnp.where(i2 == e, p2, 0.0))
                      for e in range(E)]
            rows = transpose16(rows_t)
            for t in range(E):
                out_v[pl.ds((g * E + t) * E, E)] = rows[t]
            half = lane >> 1
            even = (lane & 1) == 0
            lo = jnp.where(even, gat(i1, half), gat(i2, half))
            hi = jnp.where(even, gat(i1, 8 + half), gat(i2, 8 + half))
            idx_v[pl.ds(g * 2 * E, E)] = lo
            idx_v[pl.ds((g * 2 + 1) * E, E)] = hi
            return _

        lax.fori_loop(0, TPW // E, tile_body, None)
        co = pltpu.async_copy(
            out_v, out_hbm.at[pl.ds(wid * (TPW * E), TPW * E)], sem_o)
        ci = pltpu.async_copy(
            idx_v, idx_hbm.at[pl.ds(wid * (TPW * BEST_K), TPW * BEST_K)],
            sem_i)
        co.wait()
        ci.wait()

    return k(lg_t)


@jax.jit
def kernel(x, Wb, bb, Wn, bn):
    del Wn, bn
    lg_t = _tc_logits_t(x, Wb, bb.reshape(E, 1))
    out_flat, idx_flat = _sc_router(lg_t)
    return (out_flat.reshape(TOKENS, E), idx_flat.reshape(TOKENS, BEST_K))
